# baseline (speedup 1.0000x reference)
import jax
import jax.numpy as jnp
from jax.experimental import pallas as pl

_K = 8192
_TT = 16384
_ST = 256


def _argmax_subtile(xt, cb):
    logits = jax.lax.dot_general(
        xt, cb, (((1,), (1,)), ((), ())), preferred_element_type=jnp.float32
    )
    nc = _K // 128
    best_val = logits[:, 0:128]
    best_c = jnp.zeros((_ST, 128), jnp.float32)
    for c in range(1, nc):
        chunk = logits[:, c * 128 : (c + 1) * 128]
        pred = chunk > best_val
        best_c = jnp.where(pred, jnp.float32(c), best_c)
        best_val = jnp.maximum(chunk, best_val)
    m = jnp.max(best_val, axis=1, keepdims=True)
    lane = jax.lax.broadcasted_iota(jnp.int32, (_ST, 128), 1).astype(jnp.float32)
    k_full = best_c * 128.0 + lane
    cand = jnp.where(best_val == m, k_full, jnp.float32(_K))
    return jnp.min(cand, axis=1).astype(jnp.int32)


def _lfq_tile(x_ref, cb_ref, out_ref):
    cb = cb_ref[...]
    for s in range(_TT // _ST):
        xt = x_ref[0, s * _ST : (s + 1) * _ST]
        idx = _argmax_subtile(xt, cb)
        base = s * _ST
        if _ST >= 512:
            out_ref[0, base // 512 : (base + _ST) // 512, :] = idx.reshape(
                _ST // 512, 512
            )
        else:
            col = base % 512
            out_ref[0, base // 512, col : col + _ST] = idx


def kernel(x, codebook):
    B, T, D = x.shape
    n = (B * T) // _TT
    xf = x.reshape(n, _TT, D)
    out = pl.pallas_call(
        _lfq_tile,
        grid=(n,),
        in_specs=[
            pl.BlockSpec((1, _TT, D), lambda i: (i, 0, 0)),
            pl.BlockSpec((_K, D), lambda i: (0, 0)),
        ],
        out_specs=pl.BlockSpec((1, _TT // 512, 512), lambda i: (i, 0, 0)),
        out_shape=jax.ShapeDtypeStruct((n, _TT // 512, 512), jnp.int32),
    )(xf, codebook)
    return out.reshape(B, T), jnp.asarray(0.0, dtype=jnp.float32)


# speedup vs baseline: 1.0362x; 1.0362x over previous
import jax
import jax.numpy as jnp
from jax.experimental import pallas as pl

_K = 8192
_TT = 4096
_ST = 256


def _argmax_subtile(xt, cb):
    logits = jax.lax.dot_general(
        xt, cb, (((1,), (1,)), ((), ())), preferred_element_type=jnp.float32
    )
    nc = _K // 128
    best_val = logits[:, 0:128]
    best_c = jnp.zeros((_ST, 128), jnp.float32)
    for c in range(1, nc):
        chunk = logits[:, c * 128 : (c + 1) * 128]
        pred = chunk > best_val
        best_c = jnp.where(pred, jnp.float32(c), best_c)
        best_val = jnp.maximum(chunk, best_val)
    m = jnp.max(best_val, axis=1, keepdims=True)
    lane = jax.lax.broadcasted_iota(jnp.int32, (_ST, 128), 1).astype(jnp.float32)
    k_full = best_c * 128.0 + lane
    cand = jnp.where(best_val == m, k_full, jnp.float32(_K))
    return jnp.min(cand, axis=1).astype(jnp.int32)


def _lfq_tile(x_ref, cb_ref, out_ref):
    cb = cb_ref[...]
    for s in range(_TT // _ST):
        xt = x_ref[0, s * _ST : (s + 1) * _ST]
        idx = _argmax_subtile(xt, cb)
        base = s * _ST
        if _ST >= 512:
            out_ref[0, base // 512 : (base + _ST) // 512, :] = idx.reshape(
                _ST // 512, 512
            )
        else:
            col = base % 512
            out_ref[0, base // 512, col : col + _ST] = idx


def kernel(x, codebook):
    B, T, D = x.shape
    n = (B * T) // _TT
    xf = x.reshape(n, _TT, D)
    out = pl.pallas_call(
        _lfq_tile,
        grid=(n,),
        in_specs=[
            pl.BlockSpec((1, _TT, D), lambda i: (i, 0, 0)),
            pl.BlockSpec((_K, D), lambda i: (0, 0)),
        ],
        out_specs=pl.BlockSpec((1, _TT // 512, 512), lambda i: (i, 0, 0)),
        out_shape=jax.ShapeDtypeStruct((n, _TT // 512, 512), jnp.int32),
    )(xf, codebook)
    return out.reshape(B, T), jnp.asarray(0.0, dtype=jnp.float32)
